# Initial kernel scaffold; baseline (speedup 1.0000x reference)
import jax
import jax.numpy as jnp
from jax.experimental import pallas as pl


def kernel(pred, target):
    raise NotImplementedError("write your pallas kernel here")


# speedup vs baseline: 2.5841x; 2.5841x over previous
import functools

import jax
import jax.numpy as jnp
from jax.experimental import pallas as pl

_SMOOTHING = 0.1
_CONFIDENCE = 1.0 - _SMOOTHING


def _loss_kernel(pred_ref, target_ref, out_ref, *, num_classes, num_rows):
    x = pred_ref[...]
    t = target_ref[0, 0, :]
    m = jnp.max(x, axis=-1, keepdims=True)
    s = jnp.sum(jnp.exp(x - m), axis=-1)
    row_sum = jnp.sum(x, axis=-1)
    cols = jax.lax.broadcasted_iota(jnp.int32, x.shape, 1)
    p_t = jnp.sum(jnp.where(cols == t[:, None], x, 0.0), axis=-1)
    lse = m[:, 0] + jnp.log(s)

    sv = _SMOOTHING / (num_classes - 1)
    loss_rows = -(sv * (row_sum - num_classes * lse)
                  + (_CONFIDENCE - sv) * (p_t - lse))
    block = (jnp.sum(loss_rows) / num_rows).reshape(1, 1)

    @pl.when(pl.program_id(0) == 0)
    def _init():
        out_ref[...] = jnp.zeros((1, 1), jnp.float32)

    out_ref[...] += block


def kernel(pred, target):
    num_rows, num_classes = pred.shape
    block_rows = 32
    n_blocks = num_rows // block_rows
    target3 = target.astype(jnp.int32).reshape(n_blocks, 1, block_rows)

    out = pl.pallas_call(
        functools.partial(_loss_kernel, num_classes=num_classes,
                          num_rows=num_rows),
        grid=(n_blocks,),
        in_specs=[
            pl.BlockSpec((block_rows, num_classes), lambda i: (i, 0)),
            pl.BlockSpec((1, 1, block_rows), lambda i: (i, 0, 0)),
        ],
        out_specs=pl.BlockSpec((1, 1), lambda i: (0, 0)),
        out_shape=jax.ShapeDtypeStruct((1, 1), jnp.float32),
    )(pred, target3)
    return out[0, 0]
